# baseline (device time: 44383 ns/iter reference)
import jax
import jax.numpy as jnp
from jax import lax
from jax.experimental import pallas as pl
from jax.experimental.pallas import tpu as pltpu

NC = 4


def kernel(x):
    m, n = x.shape
    q = m // 4
    qq = q // NC

    def body(x_ref, out_ref, comm, send_sems, recv_sems):
        my_x = lax.axis_index("x")
        my_y = lax.axis_index("y")
        y_nbr = (my_x, 1 - my_y)
        x_nbr = (1 - my_x, my_y)

        a_keep = my_x * q
        a_send = (1 - my_x) * q
        b_keep = 2 * q + my_y * q
        b_send = 2 * q + (1 - my_y) * q

        barrier_sem = pltpu.get_barrier_semaphore()
        for nbr in (y_nbr, x_nbr):
            pl.semaphore_signal(
                barrier_sem, inc=1,
                device_id=nbr, device_id_type=pl.DeviceIdType.MESH,
            )
        pl.semaphore_wait(barrier_sem, 2)

        def exch(slot, src_off, dst, nbr):
            r = pltpu.make_async_remote_copy(
                src_ref=out_ref.at[pl.ds(src_off, qq), :],
                dst_ref=dst,
                send_sem=send_sems.at[slot], recv_sem=recv_sems.at[slot],
                device_id=nbr, device_id_type=pl.DeviceIdType.MESH,
            )
            r.start()
            return r

        def rs(slot, src_off, nbr):
            return exch(slot, src_off, comm.at[slot], nbr)

        def ag(slot, src_off, nbr):
            return exch(slot, src_off, out_ref.at[pl.ds(src_off, qq), :], nbr)

        def cast(off):
            out_ref[pl.ds(off, qq), :] = (
                x_ref[pl.ds(off, qq), :].astype(jnp.bfloat16)
            )

        def accum(off, slot):
            out_ref[pl.ds(off, qq), :] = (
                out_ref[pl.ds(off, qq), :] + comm[slot]
            )

        rsa, rsb = [], []
        for c in range(NC):
            cast(a_send + c * qq)
            rsa.append(rs(c, a_send + c * qq, x_nbr))
            cast(b_send + c * qq)
            rsb.append(rs(NC + c, b_send + c * qq, y_nbr))
        for c in range(NC):
            cast(a_keep + c * qq)
            cast(b_keep + c * qq)

        exa, exb = [], []
        for c in range(NC):
            rsa[c].wait()
            accum(a_keep + c * qq, c)
            exa.append(rs(2 * NC + c, a_keep + c * qq, y_nbr))
            rsb[c].wait()
            accum(b_keep + c * qq, NC + c)
            exb.append(rs(3 * NC + c, b_keep + c * qq, x_nbr))

        aga, agb = [], []
        for c in range(NC):
            exa[c].wait()
            accum(a_keep + c * qq, 2 * NC + c)
            aga.append(ag(4 * NC + c, a_keep + c * qq, x_nbr))
            exb[c].wait()
            accum(b_keep + c * qq, 3 * NC + c)
            agb.append(ag(5 * NC + c, b_keep + c * qq, y_nbr))

        for r in aga + agb:
            r.wait()

    return pl.pallas_call(
        body,
        out_shape=jax.ShapeDtypeStruct((m, n), jnp.bfloat16),
        in_specs=[pl.BlockSpec(memory_space=pltpu.VMEM)],
        out_specs=pl.BlockSpec(memory_space=pltpu.VMEM),
        scratch_shapes=[
            pltpu.VMEM((4 * NC, qq, n), jnp.bfloat16),
            pltpu.SemaphoreType.DMA((6 * NC,)),
            pltpu.SemaphoreType.DMA((6 * NC,)),
        ],
        compiler_params=pltpu.CompilerParams(collective_id=0),
    )(x)


# device time: 44175 ns/iter; 1.0047x vs baseline; 1.0047x over previous
import jax
import jax.numpy as jnp
from jax import lax
from jax.experimental import pallas as pl
from jax.experimental.pallas import tpu as pltpu

NC = 2


def kernel(x):
    m, n = x.shape
    q = m // 4
    qq = q // NC

    def body(x_ref, out_ref, comm, send_sems, recv_sems):
        my_x = lax.axis_index("x")
        my_y = lax.axis_index("y")
        y_nbr = (my_x, 1 - my_y)
        x_nbr = (1 - my_x, my_y)

        a_keep = my_x * q
        a_send = (1 - my_x) * q
        b_keep = 2 * q + my_y * q
        b_send = 2 * q + (1 - my_y) * q

        barrier_sem = pltpu.get_barrier_semaphore()
        for nbr in (y_nbr, x_nbr):
            pl.semaphore_signal(
                barrier_sem, inc=1,
                device_id=nbr, device_id_type=pl.DeviceIdType.MESH,
            )
        pl.semaphore_wait(barrier_sem, 2)

        def exch(slot, src_off, dst, nbr):
            r = pltpu.make_async_remote_copy(
                src_ref=out_ref.at[pl.ds(src_off, qq), :],
                dst_ref=dst,
                send_sem=send_sems.at[slot], recv_sem=recv_sems.at[slot],
                device_id=nbr, device_id_type=pl.DeviceIdType.MESH,
            )
            r.start()
            return r

        def rs(slot, src_off, nbr):
            return exch(slot, src_off, comm.at[slot], nbr)

        def ag(slot, src_off, nbr):
            return exch(slot, src_off, out_ref.at[pl.ds(src_off, qq), :], nbr)

        def cast(off):
            out_ref[pl.ds(off, qq), :] = (
                x_ref[pl.ds(off, qq), :].astype(jnp.bfloat16)
            )

        def accum(off, slot):
            out_ref[pl.ds(off, qq), :] = (
                out_ref[pl.ds(off, qq), :] + comm[slot]
            )

        rsa, rsb = [], []
        for c in range(NC):
            cast(a_send + c * qq)
            rsa.append(rs(c, a_send + c * qq, x_nbr))
            cast(b_send + c * qq)
            rsb.append(rs(NC + c, b_send + c * qq, y_nbr))
        for c in range(NC):
            cast(a_keep + c * qq)
            cast(b_keep + c * qq)

        exa, exb = [], []
        for c in range(NC):
            rsa[c].wait()
            accum(a_keep + c * qq, c)
            exa.append(rs(2 * NC + c, a_keep + c * qq, y_nbr))
            rsb[c].wait()
            accum(b_keep + c * qq, NC + c)
            exb.append(rs(3 * NC + c, b_keep + c * qq, x_nbr))

        aga, agb = [], []
        for c in range(NC):
            exa[c].wait()
            accum(a_keep + c * qq, 2 * NC + c)
            aga.append(ag(4 * NC + c, a_keep + c * qq, x_nbr))
            exb[c].wait()
            accum(b_keep + c * qq, 3 * NC + c)
            agb.append(ag(5 * NC + c, b_keep + c * qq, y_nbr))

        for r in aga + agb:
            r.wait()

    return pl.pallas_call(
        body,
        out_shape=jax.ShapeDtypeStruct((m, n), jnp.bfloat16),
        in_specs=[pl.BlockSpec(memory_space=pltpu.VMEM)],
        out_specs=pl.BlockSpec(memory_space=pltpu.VMEM),
        scratch_shapes=[
            pltpu.VMEM((4 * NC, qq, n), jnp.bfloat16),
            pltpu.SemaphoreType.DMA((6 * NC,)),
            pltpu.SemaphoreType.DMA((6 * NC,)),
        ],
        compiler_params=pltpu.CompilerParams(collective_id=0),
    )(x)
